# baseline (speedup 1.0000x reference)
import jax
import jax.numpy as jnp
from jax.experimental import pallas as pl
from jax.experimental.pallas import tpu as pltpu

_CB = 2048
_RB = 200


def _fused_body(aug_ref, sel_ref, out_ref, acc_ref):
    j = pl.program_id(1)

    @pl.when(j == 0)
    def _():
        acc_ref[...] = jnp.zeros_like(acc_ref)

    e = jnp.exp(sel_ref[...]).astype(jnp.bfloat16)
    aug = aug_ref[...].astype(jnp.bfloat16)
    acc_ref[...] += jax.lax.dot_general(
        aug, e, (((0,), (0,)), ((), ())), preferred_element_type=jnp.float32
    )

    @pl.when(j == pl.num_programs(1) - 1)
    def _():
        acc = acc_ref[...]
        out_ref[...] = acc[:16, :] / acc[16:17, :]


def kernel(selections, items):
    batch, n_items = selections.shape
    _, n_samples = items.shape
    sel_t = selections.T
    aug = jnp.zeros((n_items, 24), jnp.float32)
    aug = aug.at[:, :n_samples].set(items).at[:, n_samples].set(1.0)

    out_t = pl.pallas_call(
        _fused_body,
        grid=(batch // _CB, n_items // _RB),
        in_specs=[
            pl.BlockSpec((_RB, 24), lambda k, j: (j, 0)),
            pl.BlockSpec((_RB, _CB), lambda k, j: (j, k)),
        ],
        out_specs=pl.BlockSpec((n_samples, _CB), lambda k, j: (0, k)),
        out_shape=jax.ShapeDtypeStruct((n_samples, batch), jnp.float32),
        scratch_shapes=[pltpu.VMEM((24, _CB), jnp.float32)],
    )(aug, sel_t)
    return out_t.T


# speedup vs baseline: 1.5717x; 1.5717x over previous
import jax
import jax.numpy as jnp
from jax.experimental import pallas as pl
from jax.experimental.pallas import tpu as pltpu

_RB = 40
_NBUF = 6


def _body(sel_hbm, aug_ref, out_ref, buf, acc_ref, sems):
    n_chunks = sel_hbm.shape[0] // _RB

    def start_copy(chunk, slot):
        pltpu.make_async_copy(
            sel_hbm.at[pl.ds(chunk * _RB, _RB), :],
            buf.at[slot],
            sems.at[slot],
        ).start()

    for k in range(_NBUF):
        start_copy(k, k)

    def step(i, _):
        slot = jax.lax.rem(i, _NBUF)
        pltpu.make_async_copy(
            sel_hbm.at[pl.ds(i * _RB, _RB), :],
            buf.at[slot],
            sems.at[slot],
        ).wait()
        e = jnp.exp(buf[slot]).astype(jnp.bfloat16)
        aug = aug_ref[pl.ds(i * _RB, _RB), :].astype(jnp.bfloat16)
        acc_ref[...] += jax.lax.dot_general(
            aug, e, (((0,), (0,)), ((), ())), preferred_element_type=jnp.float32
        )

        @pl.when(i + _NBUF < n_chunks)
        def _():
            start_copy(i + _NBUF, slot)

        return 0

    acc_ref[...] = jnp.zeros_like(acc_ref)
    jax.lax.fori_loop(0, n_chunks, step, 0, unroll=_NBUF)
    acc = acc_ref[...]
    out_ref[...] = acc[:16, :] / acc[16:17, :]


def kernel(selections, items):
    batch, n_items = selections.shape
    _, n_samples = items.shape
    sel_t = selections.T
    aug = jnp.zeros((n_items, 24), jnp.float32)
    aug = aug.at[:, :n_samples].set(items).at[:, n_samples].set(1.0)

    out_t = pl.pallas_call(
        _body,
        in_specs=[
            pl.BlockSpec(memory_space=pltpu.MemorySpace.HBM),
            pl.BlockSpec(memory_space=pltpu.MemorySpace.VMEM),
        ],
        out_specs=pl.BlockSpec(memory_space=pltpu.MemorySpace.VMEM),
        out_shape=jax.ShapeDtypeStruct((n_samples, batch), jnp.float32),
        scratch_shapes=[
            pltpu.VMEM((_NBUF, _RB, batch), jnp.float32),
            pltpu.VMEM((24, batch), jnp.float32),
            pltpu.SemaphoreType.DMA((_NBUF,)),
        ],
    )(sel_t, aug)
    return out_t.T
